# baseline (speedup 1.0000x reference)
import functools

import jax
import jax.numpy as jnp
from jax import lax
from jax.experimental import pallas as pl
from jax.experimental.pallas import tpu as pltpu
from jax.experimental.pallas import tpu_sc as plsc

_EPS = 1e-5
_NC = 2
_NS = 16
_NW = _NC * _NS


def _sc_gather(P, Q, dst, src, row0, nrows):
    E = dst.shape[0]
    E2 = E // 2
    H = P.shape[1]
    ept = nrows // _NS
    C = 400 if ept % 400 == 0 else 200
    nch = ept // C
    mesh = plsc.VectorSubcoreMesh(core_axis_name="c", subcore_axis_name="s")

    @functools.partial(
        pl.kernel,
        out_type=(jax.ShapeDtypeStruct((nrows, 2 * H), jnp.bfloat16),
                  jax.ShapeDtypeStruct((nrows, 2 * H), jnp.bfloat16)),
        mesh=mesh,
        scratch_types=[
            pltpu.VMEM((C,), jnp.int32),
            pltpu.VMEM((C,), jnp.int32),
            pltpu.VMEM((C, H), jnp.bfloat16),
            pltpu.VMEM((C, H), jnp.bfloat16),
            pltpu.SemaphoreType.DMA,
            pltpu.SemaphoreType.DMA,
        ],
        compiler_params=pltpu.CompilerParams(use_tc_tiling_on_sc=False),
    )
    def k(p_hbm, q_hbm, dst_hbm, src_hbm, gp_hbm, gq_hbm,
          di, si, bp, bq, sem1, sem2):
        wid = lax.axis_index("s") * _NC + lax.axis_index("c")
        half = wid // _NS
        tw = wid % _NS
        col = half * H
        ebase = half * E2 + row0 + tw * ept
        rbase = tw * ept

        @pl.loop(0, nch)
        def _(c):
            off = ebase + c * C
            pltpu.sync_copy(dst_hbm.at[pl.ds(off, C)], di)
            pltpu.sync_copy(src_hbm.at[pl.ds(off, C)], si)
            cp1 = pltpu.async_copy(p_hbm.at[di], bp, sem1)
            cp2 = pltpu.async_copy(q_hbm.at[si], bq, sem2)
            cp1.wait()
            cp2.wait()
            row = rbase + c * C
            pltpu.sync_copy(bp, gp_hbm.at[pl.ds(row, C), pl.ds(col, H)])
            pltpu.sync_copy(bq, gq_hbm.at[pl.ds(row, C), pl.ds(col, H)])

    return k(P, Q, dst, src)


def _sc_scatter_add(msg2, dst, zeros_nh, row0):
    E = dst.shape[0]
    E2 = E // 2
    nrows = msg2.shape[0]
    N, H = zeros_nh.shape
    ept = nrows // _NS
    C = 400 if ept % 400 == 0 else 200
    nch = ept // C
    rpt = N // _NS
    mesh = plsc.VectorSubcoreMesh(core_axis_name="c", subcore_axis_name="s")

    @functools.partial(
        pl.kernel,
        out_type=jax.ShapeDtypeStruct((_NC * N, H), jnp.float32),
        mesh=mesh,
        scratch_types=[
            pltpu.VMEM((C,), jnp.int32),
            pltpu.VMEM((C, H), jnp.float32),
            pltpu.VMEM((rpt, H), jnp.float32),
            pltpu.VMEM_SHARED((N, H), jnp.float32),
            pltpu.SemaphoreType.DMA,
        ],
        compiler_params=pltpu.CompilerParams(use_tc_tiling_on_sc=False),
    )
    def k(msg_hbm, dst_hbm, z_hbm, out_hbm, di, bm, stage, acc_sh, sem):
        cid = lax.axis_index("c")
        sid = lax.axis_index("s")
        r0 = sid * rpt
        pltpu.sync_copy(z_hbm.at[pl.ds(r0, rpt)], stage)
        pltpu.sync_copy(stage, acc_sh.at[pl.ds(r0, rpt)])
        plsc.subcore_barrier()

        col = cid * H
        ebase = cid * E2 + row0 + sid * ept
        rbase = sid * ept

        @pl.loop(0, nch)
        def _(c):
            off = ebase + c * C
            pltpu.sync_copy(dst_hbm.at[pl.ds(off, C)], di)
            pltpu.sync_copy(msg_hbm.at[pl.ds(rbase + c * C, C),
                                       pl.ds(col, H)], bm)
            pltpu.sync_copy(bm, acc_sh.at[di], add=True)

        plsc.subcore_barrier()
        pltpu.sync_copy(acc_sh.at[pl.ds(r0, rpt)], stage)
        pltpu.sync_copy(stage, out_hbm.at[pl.ds(cid * N + r0, rpt)])

    return k(msg2, dst, zeros_nh)


_EDGE_CHUNK2 = 4000


def _edge_pass_a(gp2, gq2, eap, w1cp, b1p, row0):
    L, W = gp2.shape
    R2 = _EDGE_CHUNK2
    G = L // R2
    blk0 = row0 // R2

    def body(gp_ref, gq_ref, ea_ref, w_ref, b_ref, pre_ref, mom_ref):
        i = pl.program_id(0)
        pre = (gp_ref[...].astype(jnp.float32)
               + gq_ref[...].astype(jnp.float32) + b_ref[...] + jnp.dot(
                   ea_ref[...], w_ref[...],
                   preferred_element_type=jnp.float32))
        pre_ref[...] = pre
        mom = jnp.stack([jnp.sum(pre, axis=0), jnp.sum(pre * pre, axis=0)])
        mom_ref[...] = jnp.where(i == 0, 0.0, mom_ref[...]) + mom

    return pl.pallas_call(
        body,
        grid=(G,),
        in_specs=[
            pl.BlockSpec((R2, W), lambda i: (i, 0)),
            pl.BlockSpec((R2, W), lambda i: (i, 0)),
            pl.BlockSpec((R2, W), lambda i: (i + blk0, 0)),
            pl.BlockSpec((W, W), lambda i: (0, 0)),
            pl.BlockSpec((1, W), lambda i: (0, 0)),
        ],
        out_specs=[
            pl.BlockSpec((R2, W), lambda i: (i, 0)),
            pl.BlockSpec((2, W), lambda i: (0, 0)),
        ],
        out_shape=[
            jax.ShapeDtypeStruct((L, W), jnp.float32),
            jax.ShapeDtypeStruct((2, W), jnp.float32),
        ],
    )(gp2, gq2, eap, w1cp, b1p)


def _paired_scale(mom, gp, bep, n):
    H = mom.shape[1] // 2
    tot = mom[:, :H] + mom[:, H:]
    m = tot[0:1, :] * (1.0 / n)
    v = tot[1:2, :] * (1.0 / n) - m * m
    m2 = jnp.concatenate([m, m], axis=1)
    v2 = jnp.concatenate([v, v], axis=1)
    s = gp * lax.rsqrt(v2 + _EPS)
    t = bep - m2 * s
    return s, t


def _edge_pass_b(pre1, mom1a, mom1b, g1p, be1p, w2d, b2p, n):
    L, W = pre1.shape
    R2 = _EDGE_CHUNK2
    G = L // R2

    def body(p_ref, ma_ref, mb_ref, g_ref, be_ref, w_ref, b_ref,
             o_ref, mom_ref):
        i = pl.program_id(0)
        s, t = _paired_scale(ma_ref[...] + mb_ref[...], g_ref[...],
                             be_ref[...], n)
        h1 = jnp.maximum(p_ref[...] * s + t, 0.0)
        pre2 = jnp.dot(h1, w_ref[...],
                       preferred_element_type=jnp.float32) + b_ref[...]
        o_ref[...] = pre2
        mom = jnp.stack([jnp.sum(pre2, axis=0), jnp.sum(pre2 * pre2, axis=0)])
        mom_ref[...] = jnp.where(i == 0, 0.0, mom_ref[...]) + mom

    small = pl.BlockSpec((1, W), lambda i: (0, 0))
    return pl.pallas_call(
        body,
        grid=(G,),
        in_specs=[
            pl.BlockSpec((R2, W), lambda i: (i, 0)),
            pl.BlockSpec((2, W), lambda i: (0, 0)),
            pl.BlockSpec((2, W), lambda i: (0, 0)),
            small, small,
            pl.BlockSpec((W, W), lambda i: (0, 0)),
            small,
        ],
        out_specs=[
            pl.BlockSpec((R2, W), lambda i: (i, 0)),
            pl.BlockSpec((2, W), lambda i: (0, 0)),
        ],
        out_shape=[
            jax.ShapeDtypeStruct((L, W), jnp.float32),
            jax.ShapeDtypeStruct((2, W), jnp.float32),
        ],
    )(pre1, mom1a, mom1b, g1p, be1p, w2d, b2p)


def _edge_pass_c(pre2, mom2a, mom2b, g2p, be2p, n):
    L, W = pre2.shape
    R2 = _EDGE_CHUNK2
    G = L // R2

    def body(p_ref, ma_ref, mb_ref, g_ref, be_ref, o_ref):
        s, t = _paired_scale(ma_ref[...] + mb_ref[...], g_ref[...],
                             be_ref[...], n)
        o_ref[...] = jnp.maximum(p_ref[...] * s + t, 0.0)

    small = pl.BlockSpec((1, W), lambda i: (0, 0))
    return pl.pallas_call(
        body,
        grid=(G,),
        in_specs=[
            pl.BlockSpec((R2, W), lambda i: (i, 0)),
            pl.BlockSpec((2, W), lambda i: (0, 0)),
            pl.BlockSpec((2, W), lambda i: (0, 0)),
            small, small,
        ],
        out_specs=pl.BlockSpec((R2, W), lambda i: (i, 0)),
        out_shape=jax.ShapeDtypeStruct((L, W), jnp.float32),
    )(pre2, mom2a, mom2b, g2p, be2p)


def _node_init(x, w_in, b_in, w1a, w1b):
    N = x.shape[0]
    H = w_in.shape[1]

    def body(x_ref, w_ref, b_ref, wa_ref, wb_ref, h_ref, p_ref, q_ref):
        h = jnp.dot(x_ref[...], w_ref[...],
                    preferred_element_type=jnp.float32) + b_ref[...]
        h_ref[...] = h
        p_ref[...] = jnp.dot(
            h, wa_ref[...],
            preferred_element_type=jnp.float32).astype(jnp.bfloat16)
        q_ref[...] = jnp.dot(
            h, wb_ref[...],
            preferred_element_type=jnp.float32).astype(jnp.bfloat16)

    return pl.pallas_call(
        body,
        out_shape=[jax.ShapeDtypeStruct((N, H), jnp.float32),
                   jax.ShapeDtypeStruct((N, H), jnp.bfloat16),
                   jax.ShapeDtypeStruct((N, H), jnp.bfloat16)],
    )(x, w_in, b_in.reshape(1, H), w1a, w1b)


def _node_update(h, parts, uw1a, uw1b, ub1, ug1, ube1, uw2, ub2,
                 ug2, ube2, w1a_next, w1b_next):
    N, H = h.shape

    def body(h_ref, p_ref, w1a_ref, w1b_ref, b1_ref, g1_ref, be1_ref,
             w2_ref, b2_ref, g2_ref, be2_ref, wan_ref, wbn_ref,
             o_ref, pn_ref, qn_ref):
        h_ = h_ref[...]
        aggr = p_ref[:N, :] + p_ref[N:, :]
        pre1 = (jnp.dot(h_, w1a_ref[...], preferred_element_type=jnp.float32)
                + jnp.dot(aggr, w1b_ref[...],
                          preferred_element_type=jnp.float32) + b1_ref[...])
        m1 = jnp.mean(pre1, axis=0, keepdims=True)
        v1 = jnp.mean((pre1 - m1) ** 2, axis=0, keepdims=True)
        h1 = jnp.maximum((pre1 - m1) * lax.rsqrt(v1 + _EPS) * g1_ref[...]
                         + be1_ref[...], 0.0)
        pre2 = jnp.dot(h1, w2_ref[...],
                       preferred_element_type=jnp.float32) + b2_ref[...]
        m2 = jnp.mean(pre2, axis=0, keepdims=True)
        v2 = jnp.mean((pre2 - m2) ** 2, axis=0, keepdims=True)
        upd = jnp.maximum((pre2 - m2) * lax.rsqrt(v2 + _EPS) * g2_ref[...]
                          + be2_ref[...], 0.0)
        h_new = h_ + upd
        o_ref[...] = h_new
        pn_ref[...] = jnp.dot(
            h_new, wan_ref[...],
            preferred_element_type=jnp.float32).astype(jnp.bfloat16)
        qn_ref[...] = jnp.dot(
            h_new, wbn_ref[...],
            preferred_element_type=jnp.float32).astype(jnp.bfloat16)

    return pl.pallas_call(
        body,
        out_shape=[jax.ShapeDtypeStruct((N, H), jnp.float32),
                   jax.ShapeDtypeStruct((N, H), jnp.bfloat16),
                   jax.ShapeDtypeStruct((N, H), jnp.bfloat16)],
    )(h, parts, uw1a, uw1b, ub1.reshape(1, H), ug1.reshape(1, H),
      ube1.reshape(1, H), uw2, ub2.reshape(1, H), ug2.reshape(1, H),
      ube2.reshape(1, H), w1a_next, w1b_next)


def _pool_project(h, batch, w_mu, b_mu, ng):
    N, H = h.shape
    LAT = w_mu.shape[1]

    def body(h_ref, b_ref, w_ref, bm_ref, o_ref):
        seg = b_ref[...]
        onehot = (seg == lax.broadcasted_iota(jnp.int32, (N, ng), 1)
                  ).astype(jnp.float32)
        sums = lax.dot_general(onehot, h_ref[...],
                               (((0,), (0,)), ((), ())),
                               preferred_element_type=jnp.float32)
        counts = jnp.sum(onehot, axis=0)[:, None]
        pooled = sums / jnp.maximum(counts, 1.0)
        o_ref[...] = jnp.dot(pooled, w_ref[...],
                             preferred_element_type=jnp.float32) + bm_ref[...]

    return pl.pallas_call(
        body,
        out_shape=jax.ShapeDtypeStruct((ng, LAT), jnp.float32),
    )(h, batch.reshape(N, 1), w_mu, b_mu.reshape(1, LAT))


def _pair(v):
    return jnp.concatenate([v, v]).reshape(1, -1)


@jax.jit
def kernel(x, edge_index, edge_attr, batch, params):
    N = x.shape[0]
    H = params['W_in'].shape[1]
    E = edge_index.shape[1]
    EDIM = edge_attr.shape[1]
    NG = 64
    src = edge_index[0]
    dst = edge_index[1]
    zeros_nh = jnp.zeros((N, H), jnp.float32)

    E2 = E // 2
    eap = jnp.pad(jnp.concatenate([edge_attr[:E2], edge_attr[E2:]], axis=1),
                  ((0, 0), (0, 2 * H - 2 * EDIM)))

    layers = params['layers']
    l0 = layers[0]['msg']
    h, P, Q = _node_init(x, params['W_in'], params['b_in'],
                         l0['W1'][:H], l0['W1'][H:2 * H])

    for li, lay in enumerate(layers):
        mp = lay['msg']
        up = lay['upd']
        w1c = mp['W1'][2 * H:]
        w1cp = jnp.zeros((2 * H, 2 * H), jnp.float32)
        w1cp = w1cp.at[:EDIM, :H].set(w1c).at[EDIM:2 * EDIM, H:].set(w1c)
        w2d = jnp.zeros((2 * H, 2 * H), jnp.float32)
        w2d = w2d.at[:H, :H].set(mp['W2']).at[H:, H:].set(mp['W2'])

        mzero = jnp.zeros((2, 2 * H), jnp.float32)
        gp2, gq2 = _sc_gather(P, Q, dst, src, 0, E2)
        pre1, mom1 = _edge_pass_a(gp2, gq2, eap, w1cp, _pair(mp['b1']), 0)
        pre2, mom2 = _edge_pass_b(pre1, mom1, mzero, _pair(mp['g1']),
                                  _pair(mp['be1']), w2d, _pair(mp['b2']), E)
        msg2 = _edge_pass_c(pre2, mom2, mzero, _pair(mp['g2']),
                            _pair(mp['be2']), E)
        parts = _sc_scatter_add(msg2, dst, zeros_nh, 0)
        if li + 1 < len(layers):
            nmp = layers[li + 1]['msg']
            wan, wbn = nmp['W1'][:H], nmp['W1'][H:2 * H]
        else:
            wan, wbn = up['W2'], up['W2']
        h, P, Q = _node_update(h, parts, up['W1'][:H],
                               up['W1'][H:2 * H], up['b1'], up['g1'],
                               up['be1'], up['W2'], up['b2'], up['g2'],
                               up['be2'], wan, wbn)

    return _pool_project(h, batch, params['W_mu'], params['b_mu'], NG)


# speedup vs baseline: 1.5492x; 1.5492x over previous
import functools

import jax
import jax.numpy as jnp
from jax import lax
from jax.experimental import pallas as pl
from jax.experimental.pallas import tpu as pltpu
from jax.experimental.pallas import tpu_sc as plsc

_EPS = 1e-5
_NC = 2
_NS = 16
_NW = _NC * _NS


def _sc_gather(P, Q, dst, src, row0, nrows):
    E = dst.shape[0]
    E2 = E // 2
    H = P.shape[1]
    ept = nrows // _NS
    C = 400 if ept % 400 == 0 else 200
    nch = ept // C
    mesh = plsc.VectorSubcoreMesh(core_axis_name="c", subcore_axis_name="s")

    @functools.partial(
        pl.kernel,
        out_type=(jax.ShapeDtypeStruct((nrows, 2 * H), jnp.float32),
                  jax.ShapeDtypeStruct((nrows, 2 * H), jnp.float32)),
        mesh=mesh,
        scratch_types=[
            pltpu.VMEM((C,), jnp.int32),
            pltpu.VMEM((C,), jnp.int32),
            pltpu.VMEM((C, H), jnp.float32),
            pltpu.VMEM((C, H), jnp.float32),
            pltpu.SemaphoreType.DMA,
            pltpu.SemaphoreType.DMA,
        ],
        compiler_params=pltpu.CompilerParams(use_tc_tiling_on_sc=False),
    )
    def k(p_hbm, q_hbm, dst_hbm, src_hbm, gp_hbm, gq_hbm,
          di, si, bp, bq, sem1, sem2):
        wid = lax.axis_index("s") * _NC + lax.axis_index("c")
        half = wid // _NS
        tw = wid % _NS
        col = half * H
        ebase = half * E2 + row0 + tw * ept
        rbase = tw * ept

        @pl.loop(0, nch)
        def _(c):
            off = ebase + c * C
            pltpu.sync_copy(dst_hbm.at[pl.ds(off, C)], di)
            pltpu.sync_copy(src_hbm.at[pl.ds(off, C)], si)
            cp1 = pltpu.async_copy(p_hbm.at[di], bp, sem1)
            cp2 = pltpu.async_copy(q_hbm.at[si], bq, sem2)
            cp1.wait()
            cp2.wait()
            row = rbase + c * C
            pltpu.sync_copy(bp, gp_hbm.at[pl.ds(row, C), pl.ds(col, H)])
            pltpu.sync_copy(bq, gq_hbm.at[pl.ds(row, C), pl.ds(col, H)])

    return k(P, Q, dst, src)


def _sc_scatter_add(msg2, dst, zeros_nh, row0):
    E = dst.shape[0]
    E2 = E // 2
    nrows = msg2.shape[0]
    N, H = zeros_nh.shape
    ept = nrows // _NS
    C = 400 if ept % 400 == 0 else 200
    nch = ept // C
    rpt = N // _NS
    mesh = plsc.VectorSubcoreMesh(core_axis_name="c", subcore_axis_name="s")

    @functools.partial(
        pl.kernel,
        out_type=jax.ShapeDtypeStruct((_NC * N, H), jnp.float32),
        mesh=mesh,
        scratch_types=[
            pltpu.VMEM((C,), jnp.int32),
            pltpu.VMEM((C, H), jnp.float32),
            pltpu.VMEM((rpt, H), jnp.float32),
            pltpu.VMEM_SHARED((N, H), jnp.float32),
            pltpu.SemaphoreType.DMA,
        ],
        compiler_params=pltpu.CompilerParams(use_tc_tiling_on_sc=False),
    )
    def k(msg_hbm, dst_hbm, z_hbm, out_hbm, di, bm, stage, acc_sh, sem):
        cid = lax.axis_index("c")
        sid = lax.axis_index("s")
        r0 = sid * rpt
        pltpu.sync_copy(z_hbm.at[pl.ds(r0, rpt)], stage)
        pltpu.sync_copy(stage, acc_sh.at[pl.ds(r0, rpt)])
        plsc.subcore_barrier()

        col = cid * H
        ebase = cid * E2 + row0 + sid * ept
        rbase = sid * ept

        @pl.loop(0, nch)
        def _(c):
            off = ebase + c * C
            pltpu.sync_copy(dst_hbm.at[pl.ds(off, C)], di)
            pltpu.sync_copy(msg_hbm.at[pl.ds(rbase + c * C, C),
                                       pl.ds(col, H)], bm)
            pltpu.sync_copy(bm, acc_sh.at[di], add=True)

        plsc.subcore_barrier()
        pltpu.sync_copy(acc_sh.at[pl.ds(r0, rpt)], stage)
        pltpu.sync_copy(stage, out_hbm.at[pl.ds(cid * N + r0, rpt)])

    return k(msg2, dst, zeros_nh)


_EDGE_CHUNK2 = 4000


def _edge_pass_a(gp2, gq2, eap, w1cp, b1p, row0):
    L, W = gp2.shape
    R2 = _EDGE_CHUNK2
    G = L // R2
    blk0 = row0 // R2

    def body(gp_ref, gq_ref, ea_ref, w_ref, b_ref, pre_ref, mom_ref):
        i = pl.program_id(0)
        pre = gp_ref[...] + gq_ref[...] + b_ref[...] + jnp.dot(
            ea_ref[...], w_ref[...], preferred_element_type=jnp.float32)
        pre_ref[...] = pre
        mom = jnp.stack([jnp.sum(pre, axis=0), jnp.sum(pre * pre, axis=0)])
        mom_ref[...] = jnp.where(i == 0, 0.0, mom_ref[...]) + mom

    return pl.pallas_call(
        body,
        grid=(G,),
        in_specs=[
            pl.BlockSpec((R2, W), lambda i: (i, 0)),
            pl.BlockSpec((R2, W), lambda i: (i, 0)),
            pl.BlockSpec((R2, W), lambda i: (i + blk0, 0)),
            pl.BlockSpec((W, W), lambda i: (0, 0)),
            pl.BlockSpec((1, W), lambda i: (0, 0)),
        ],
        out_specs=[
            pl.BlockSpec((R2, W), lambda i: (i, 0)),
            pl.BlockSpec((2, W), lambda i: (0, 0)),
        ],
        out_shape=[
            jax.ShapeDtypeStruct((L, W), jnp.float32),
            jax.ShapeDtypeStruct((2, W), jnp.float32),
        ],
    )(gp2, gq2, eap, w1cp, b1p)


def _paired_scale(mom, gp, bep, n):
    H = mom.shape[1] // 2
    tot = mom[:, :H] + mom[:, H:]
    m = tot[0:1, :] * (1.0 / n)
    v = tot[1:2, :] * (1.0 / n) - m * m
    m2 = jnp.concatenate([m, m], axis=1)
    v2 = jnp.concatenate([v, v], axis=1)
    s = gp * lax.rsqrt(v2 + _EPS)
    t = bep - m2 * s
    return s, t


def _edge_pass_b(pre1, mom1a, mom1b, g1p, be1p, w2d, b2p, n):
    L, W = pre1.shape
    R2 = _EDGE_CHUNK2
    G = L // R2

    def body(p_ref, ma_ref, mb_ref, g_ref, be_ref, w_ref, b_ref,
             o_ref, mom_ref):
        i = pl.program_id(0)
        s, t = _paired_scale(ma_ref[...] + mb_ref[...], g_ref[...],
                             be_ref[...], n)
        h1 = jnp.maximum(p_ref[...] * s + t, 0.0)
        pre2 = jnp.dot(h1.astype(jnp.bfloat16), w_ref[...],
                       preferred_element_type=jnp.float32) + b_ref[...]
        o_ref[...] = pre2
        mom = jnp.stack([jnp.sum(pre2, axis=0), jnp.sum(pre2 * pre2, axis=0)])
        mom_ref[...] = jnp.where(i == 0, 0.0, mom_ref[...]) + mom

    small = pl.BlockSpec((1, W), lambda i: (0, 0))
    return pl.pallas_call(
        body,
        grid=(G,),
        in_specs=[
            pl.BlockSpec((R2, W), lambda i: (i, 0)),
            pl.BlockSpec((2, W), lambda i: (0, 0)),
            pl.BlockSpec((2, W), lambda i: (0, 0)),
            small, small,
            pl.BlockSpec((W, W), lambda i: (0, 0)),
            small,
        ],
        out_specs=[
            pl.BlockSpec((R2, W), lambda i: (i, 0)),
            pl.BlockSpec((2, W), lambda i: (0, 0)),
        ],
        out_shape=[
            jax.ShapeDtypeStruct((L, W), jnp.float32),
            jax.ShapeDtypeStruct((2, W), jnp.float32),
        ],
    )(pre1, mom1a, mom1b, g1p, be1p, w2d, b2p)


def _edge_pass_c(pre2, mom2a, mom2b, g2p, be2p, n):
    L, W = pre2.shape
    R2 = _EDGE_CHUNK2
    G = L // R2

    def body(p_ref, ma_ref, mb_ref, g_ref, be_ref, o_ref):
        s, t = _paired_scale(ma_ref[...] + mb_ref[...], g_ref[...],
                             be_ref[...], n)
        o_ref[...] = jnp.maximum(p_ref[...] * s + t, 0.0)

    small = pl.BlockSpec((1, W), lambda i: (0, 0))
    return pl.pallas_call(
        body,
        grid=(G,),
        in_specs=[
            pl.BlockSpec((R2, W), lambda i: (i, 0)),
            pl.BlockSpec((2, W), lambda i: (0, 0)),
            pl.BlockSpec((2, W), lambda i: (0, 0)),
            small, small,
        ],
        out_specs=pl.BlockSpec((R2, W), lambda i: (i, 0)),
        out_shape=jax.ShapeDtypeStruct((L, W), jnp.float32),
    )(pre2, mom2a, mom2b, g2p, be2p)


def _node_init(x, w_in, b_in, w1a, w1b):
    N = x.shape[0]
    H = w_in.shape[1]

    def body(x_ref, w_ref, b_ref, wa_ref, wb_ref, h_ref, p_ref, q_ref):
        h = jnp.dot(x_ref[...], w_ref[...],
                    preferred_element_type=jnp.float32) + b_ref[...]
        h_ref[...] = h
        p_ref[...] = jnp.dot(h, wa_ref[...], preferred_element_type=jnp.float32)
        q_ref[...] = jnp.dot(h, wb_ref[...], preferred_element_type=jnp.float32)

    return pl.pallas_call(
        body,
        out_shape=[jax.ShapeDtypeStruct((N, H), jnp.float32)] * 3,
    )(x, w_in, b_in.reshape(1, H), w1a, w1b)


def _node_update(h, parts, uw1a, uw1b, ub1, ug1, ube1, uw2, ub2,
                 ug2, ube2, w1a_next, w1b_next):
    N, H = h.shape

    def body(h_ref, p_ref, w1a_ref, w1b_ref, b1_ref, g1_ref, be1_ref,
             w2_ref, b2_ref, g2_ref, be2_ref, wan_ref, wbn_ref,
             o_ref, pn_ref, qn_ref):
        h_ = h_ref[...]
        aggr = p_ref[:N, :] + p_ref[N:, :]
        pre1 = (jnp.dot(h_, w1a_ref[...], preferred_element_type=jnp.float32)
                + jnp.dot(aggr, w1b_ref[...],
                          preferred_element_type=jnp.float32) + b1_ref[...])
        m1 = jnp.mean(pre1, axis=0, keepdims=True)
        v1 = jnp.mean((pre1 - m1) ** 2, axis=0, keepdims=True)
        h1 = jnp.maximum((pre1 - m1) * lax.rsqrt(v1 + _EPS) * g1_ref[...]
                         + be1_ref[...], 0.0)
        pre2 = jnp.dot(h1, w2_ref[...],
                       preferred_element_type=jnp.float32) + b2_ref[...]
        m2 = jnp.mean(pre2, axis=0, keepdims=True)
        v2 = jnp.mean((pre2 - m2) ** 2, axis=0, keepdims=True)
        upd = jnp.maximum((pre2 - m2) * lax.rsqrt(v2 + _EPS) * g2_ref[...]
                          + be2_ref[...], 0.0)
        h_new = h_ + upd
        o_ref[...] = h_new
        pn_ref[...] = jnp.dot(h_new, wan_ref[...],
                              preferred_element_type=jnp.float32)
        qn_ref[...] = jnp.dot(h_new, wbn_ref[...],
                              preferred_element_type=jnp.float32)

    return pl.pallas_call(
        body,
        out_shape=[jax.ShapeDtypeStruct((N, H), jnp.float32)] * 3,
    )(h, parts, uw1a, uw1b, ub1.reshape(1, H), ug1.reshape(1, H),
      ube1.reshape(1, H), uw2, ub2.reshape(1, H), ug2.reshape(1, H),
      ube2.reshape(1, H), w1a_next, w1b_next)


def _pool_project(h, batch, w_mu, b_mu, ng):
    N, H = h.shape
    LAT = w_mu.shape[1]

    def body(h_ref, b_ref, w_ref, bm_ref, o_ref):
        seg = b_ref[...]
        onehot = (seg == lax.broadcasted_iota(jnp.int32, (N, ng), 1)
                  ).astype(jnp.float32)
        sums = lax.dot_general(onehot, h_ref[...],
                               (((0,), (0,)), ((), ())),
                               preferred_element_type=jnp.float32)
        counts = jnp.sum(onehot, axis=0)[:, None]
        pooled = sums / jnp.maximum(counts, 1.0)
        o_ref[...] = jnp.dot(pooled, w_ref[...],
                             preferred_element_type=jnp.float32) + bm_ref[...]

    return pl.pallas_call(
        body,
        out_shape=jax.ShapeDtypeStruct((ng, LAT), jnp.float32),
    )(h, batch.reshape(N, 1), w_mu, b_mu.reshape(1, LAT))


def _pair(v):
    return jnp.concatenate([v, v]).reshape(1, -1)


@jax.jit
def kernel(x, edge_index, edge_attr, batch, params):
    N = x.shape[0]
    H = params['W_in'].shape[1]
    E = edge_index.shape[1]
    EDIM = edge_attr.shape[1]
    NG = 64
    src = edge_index[0]
    dst = edge_index[1]
    zeros_nh = jnp.zeros((N, H), jnp.float32)

    E2 = E // 2
    eap = jnp.pad(jnp.concatenate([edge_attr[:E2], edge_attr[E2:]], axis=1),
                  ((0, 0), (0, 2 * H - 2 * EDIM))).astype(jnp.bfloat16)

    layers = params['layers']
    l0 = layers[0]['msg']
    h, P, Q = _node_init(x, params['W_in'], params['b_in'],
                         l0['W1'][:H], l0['W1'][H:2 * H])

    for li, lay in enumerate(layers):
        mp = lay['msg']
        up = lay['upd']
        w1c = mp['W1'][2 * H:]
        w1cp = jnp.zeros((2 * H, 2 * H), jnp.float32)
        w1cp = w1cp.at[:EDIM, :H].set(w1c).at[EDIM:2 * EDIM, H:].set(w1c)
        w2d = jnp.zeros((2 * H, 2 * H), jnp.float32)
        w2d = w2d.at[:H, :H].set(mp['W2']).at[H:, H:].set(mp['W2'])

        mzero = jnp.zeros((2, 2 * H), jnp.float32)
        gp2, gq2 = _sc_gather(P, Q, dst, src, 0, E2)
        pre1, mom1 = _edge_pass_a(gp2, gq2, eap,
                                  w1cp.astype(jnp.bfloat16),
                                  _pair(mp['b1']), 0)
        pre2, mom2 = _edge_pass_b(pre1, mom1, mzero, _pair(mp['g1']),
                                  _pair(mp['be1']),
                                  w2d.astype(jnp.bfloat16),
                                  _pair(mp['b2']), E)
        msg2 = _edge_pass_c(pre2, mom2, mzero, _pair(mp['g2']),
                            _pair(mp['be2']), E)
        parts = _sc_scatter_add(msg2, dst, zeros_nh, 0)
        if li + 1 < len(layers):
            nmp = layers[li + 1]['msg']
            wan, wbn = nmp['W1'][:H], nmp['W1'][H:2 * H]
        else:
            wan, wbn = up['W2'], up['W2']
        h, P, Q = _node_update(h, parts, up['W1'][:H],
                               up['W1'][H:2 * H], up['b1'], up['g1'],
                               up['be1'], up['W2'], up['b2'], up['g2'],
                               up['be2'], wan, wbn)

    return _pool_project(h, batch, params['W_mu'], params['b_mu'], NG)


# speedup vs baseline: 1.7550x; 1.1328x over previous
import functools

import jax
import jax.numpy as jnp
from jax import lax
from jax.experimental import pallas as pl
from jax.experimental.pallas import tpu as pltpu
from jax.experimental.pallas import tpu_sc as plsc

_EPS = 1e-5
_NC = 2
_NS = 16
_NW = _NC * _NS


def _sc_gather(P, Q, dst, src, row0, nrows):
    E = dst.shape[0]
    E2 = E // 2
    H = P.shape[1]
    ept = nrows // _NS
    C = 400 if ept % 400 == 0 else 200
    nch = ept // C
    mesh = plsc.VectorSubcoreMesh(core_axis_name="c", subcore_axis_name="s")

    @functools.partial(
        pl.kernel,
        out_type=(jax.ShapeDtypeStruct((nrows, 2 * H), jnp.float32),
                  jax.ShapeDtypeStruct((nrows, 2 * H), jnp.float32)),
        mesh=mesh,
        scratch_types=[
            pltpu.VMEM((C,), jnp.int32),
            pltpu.VMEM((C,), jnp.int32),
            pltpu.VMEM((C, H), jnp.float32),
            pltpu.VMEM((C, H), jnp.float32),
            pltpu.SemaphoreType.DMA,
            pltpu.SemaphoreType.DMA,
        ],
        compiler_params=pltpu.CompilerParams(use_tc_tiling_on_sc=False),
    )
    def k(p_hbm, q_hbm, dst_hbm, src_hbm, gp_hbm, gq_hbm,
          di, si, bp, bq, sem1, sem2):
        wid = lax.axis_index("s") * _NC + lax.axis_index("c")
        half = wid // _NS
        tw = wid % _NS
        col = half * H
        ebase = half * E2 + row0 + tw * ept
        rbase = tw * ept

        @pl.loop(0, nch)
        def _(c):
            off = ebase + c * C
            pltpu.sync_copy(dst_hbm.at[pl.ds(off, C)], di)
            pltpu.sync_copy(src_hbm.at[pl.ds(off, C)], si)
            cp1 = pltpu.async_copy(p_hbm.at[di], bp, sem1)
            cp2 = pltpu.async_copy(q_hbm.at[si], bq, sem2)
            cp1.wait()
            cp2.wait()
            row = rbase + c * C
            pltpu.sync_copy(bp, gp_hbm.at[pl.ds(row, C), pl.ds(col, H)])
            pltpu.sync_copy(bq, gq_hbm.at[pl.ds(row, C), pl.ds(col, H)])

    return k(P, Q, dst, src)


def _sc_gather_sum(P, Q, dst, src):
    E = dst.shape[0]
    E2 = E // 2
    H = P.shape[1]
    ept = E // _NW
    C = 400
    nch = ept // C
    mesh = plsc.VectorSubcoreMesh(core_axis_name="c", subcore_axis_name="s")

    @functools.partial(
        pl.kernel,
        out_type=jax.ShapeDtypeStruct((E2, 2 * H), jnp.float32),
        mesh=mesh,
        scratch_types=[
            pltpu.VMEM((C,), jnp.int32),
            pltpu.VMEM((C,), jnp.int32),
            pltpu.VMEM((C,), jnp.int32),
            pltpu.VMEM((C,), jnp.int32),
            pltpu.VMEM((C, H), jnp.float32),
            pltpu.VMEM((C, H), jnp.float32),
            pltpu.VMEM((C, H), jnp.float32),
            pltpu.VMEM((C, H), jnp.float32),
            pltpu.SemaphoreType.DMA,
            pltpu.SemaphoreType.DMA,
            pltpu.SemaphoreType.DMA,
        ],
        compiler_params=pltpu.CompilerParams(use_tc_tiling_on_sc=False),
    )
    def k(p_hbm, q_hbm, dst_hbm, src_hbm, g_hbm,
          di0, si0, di1, si1, a0, b0, a1, b1, sg0, sg1, sw):
        wid = lax.axis_index("s") * _NC + lax.axis_index("c")
        half = wid // _NS
        tw = wid % _NS
        col = half * H
        ebase = half * E2 + tw * ept
        rbase = tw * ept
        bufs = [(di0, si0, a0, b0, sg0), (di1, si1, a1, b1, sg1)]

        def issue(c):
            di, si, a, b, sg = bufs[c % 2]
            off = ebase + c * C
            pltpu.sync_copy(dst_hbm.at[pl.ds(off, C)], di)
            pltpu.sync_copy(src_hbm.at[pl.ds(off, C)], si)
            return (pltpu.async_copy(p_hbm.at[di], a, sg),
                    pltpu.async_copy(q_hbm.at[si], b, sg))

        pend = {0: issue(0), 1: issue(1)}
        wr = {}
        for c in range(nch):
            g1, g2 = pend.pop(c)
            g1.wait()
            g2.wait()
            _, _, a, b, _ = bufs[c % 2]

            @pl.loop(0, C)
            def _(r):
                for kk in range(H // 16):
                    sl = pl.ds(kk * 16, 16)
                    a[r, sl] = a[r, sl] + b[r, sl]

            wr[c] = pltpu.async_copy(
                a, g_hbm.at[pl.ds(rbase + c * C, C), pl.ds(col, H)], sw)
            nxt = c + 2
            if nxt < nch:
                wr.pop(c).wait()
                pend[nxt] = issue(nxt)
        for c in sorted(wr):
            wr.pop(c).wait()

    return k(P, Q, dst, src)


def _sc_scatter_add(msg2, dst, zeros_nh, row0):
    E = dst.shape[0]
    E2 = E // 2
    nrows = msg2.shape[0]
    N, H = zeros_nh.shape
    ept = nrows // _NS
    C = 400 if ept % 400 == 0 else 200
    nch = ept // C
    rpt = N // _NS
    mesh = plsc.VectorSubcoreMesh(core_axis_name="c", subcore_axis_name="s")

    @functools.partial(
        pl.kernel,
        out_type=jax.ShapeDtypeStruct((_NC * N, H), jnp.float32),
        mesh=mesh,
        scratch_types=[
            pltpu.VMEM((C,), jnp.int32),
            pltpu.VMEM((C, H), jnp.float32),
            pltpu.VMEM((rpt, H), jnp.float32),
            pltpu.VMEM_SHARED((N, H), jnp.float32),
            pltpu.SemaphoreType.DMA,
        ],
        compiler_params=pltpu.CompilerParams(use_tc_tiling_on_sc=False),
    )
    def k(msg_hbm, dst_hbm, z_hbm, out_hbm, di, bm, stage, acc_sh, sem):
        cid = lax.axis_index("c")
        sid = lax.axis_index("s")
        r0 = sid * rpt
        pltpu.sync_copy(z_hbm.at[pl.ds(r0, rpt)], stage)
        pltpu.sync_copy(stage, acc_sh.at[pl.ds(r0, rpt)])
        plsc.subcore_barrier()

        col = cid * H
        ebase = cid * E2 + row0 + sid * ept
        rbase = sid * ept

        @pl.loop(0, nch)
        def _(c):
            off = ebase + c * C
            pltpu.sync_copy(dst_hbm.at[pl.ds(off, C)], di)
            pltpu.sync_copy(msg_hbm.at[pl.ds(rbase + c * C, C),
                                       pl.ds(col, H)], bm)
            pltpu.sync_copy(bm, acc_sh.at[di], add=True)

        plsc.subcore_barrier()
        pltpu.sync_copy(acc_sh.at[pl.ds(r0, rpt)], stage)
        pltpu.sync_copy(stage, out_hbm.at[pl.ds(cid * N + r0, rpt)])

    return k(msg2, dst, zeros_nh)


_EDGE_CHUNK2 = 4000


def _edge_pass_a(g2, eap, w1cp, b1p, row0):
    L, W = g2.shape
    R2 = _EDGE_CHUNK2
    G = L // R2
    blk0 = row0 // R2

    def body(g_ref, ea_ref, w_ref, b_ref, pre_ref, mom_ref):
        i = pl.program_id(0)
        pre = g_ref[...] + b_ref[...] + jnp.dot(
            ea_ref[...], w_ref[...], preferred_element_type=jnp.float32)
        pre_ref[...] = pre
        mom = jnp.stack([jnp.sum(pre, axis=0), jnp.sum(pre * pre, axis=0)])
        mom_ref[...] = jnp.where(i == 0, 0.0, mom_ref[...]) + mom

    return pl.pallas_call(
        body,
        grid=(G,),
        in_specs=[
            pl.BlockSpec((R2, W), lambda i: (i, 0)),
            pl.BlockSpec((R2, W), lambda i: (i + blk0, 0)),
            pl.BlockSpec((W, W), lambda i: (0, 0)),
            pl.BlockSpec((1, W), lambda i: (0, 0)),
        ],
        out_specs=[
            pl.BlockSpec((R2, W), lambda i: (i, 0)),
            pl.BlockSpec((2, W), lambda i: (0, 0)),
        ],
        out_shape=[
            jax.ShapeDtypeStruct((L, W), jnp.float32),
            jax.ShapeDtypeStruct((2, W), jnp.float32),
        ],
    )(g2, eap, w1cp, b1p)


def _paired_scale(mom, gp, bep, n):
    H = mom.shape[1] // 2
    tot = mom[:, :H] + mom[:, H:]
    m = tot[0:1, :] * (1.0 / n)
    v = tot[1:2, :] * (1.0 / n) - m * m
    m2 = jnp.concatenate([m, m], axis=1)
    v2 = jnp.concatenate([v, v], axis=1)
    s = gp * lax.rsqrt(v2 + _EPS)
    t = bep - m2 * s
    return s, t


def _edge_pass_b(pre1, mom1a, mom1b, g1p, be1p, w2d, b2p, n):
    L, W = pre1.shape
    R2 = _EDGE_CHUNK2
    G = L // R2

    def body(p_ref, ma_ref, mb_ref, g_ref, be_ref, w_ref, b_ref,
             o_ref, mom_ref):
        i = pl.program_id(0)
        s, t = _paired_scale(ma_ref[...] + mb_ref[...], g_ref[...],
                             be_ref[...], n)
        h1 = jnp.maximum(p_ref[...] * s + t, 0.0)
        pre2 = jnp.dot(h1.astype(jnp.bfloat16), w_ref[...],
                       preferred_element_type=jnp.float32) + b_ref[...]
        o_ref[...] = pre2
        mom = jnp.stack([jnp.sum(pre2, axis=0), jnp.sum(pre2 * pre2, axis=0)])
        mom_ref[...] = jnp.where(i == 0, 0.0, mom_ref[...]) + mom

    small = pl.BlockSpec((1, W), lambda i: (0, 0))
    return pl.pallas_call(
        body,
        grid=(G,),
        in_specs=[
            pl.BlockSpec((R2, W), lambda i: (i, 0)),
            pl.BlockSpec((2, W), lambda i: (0, 0)),
            pl.BlockSpec((2, W), lambda i: (0, 0)),
            small, small,
            pl.BlockSpec((W, W), lambda i: (0, 0)),
            small,
        ],
        out_specs=[
            pl.BlockSpec((R2, W), lambda i: (i, 0)),
            pl.BlockSpec((2, W), lambda i: (0, 0)),
        ],
        out_shape=[
            jax.ShapeDtypeStruct((L, W), jnp.float32),
            jax.ShapeDtypeStruct((2, W), jnp.float32),
        ],
    )(pre1, mom1a, mom1b, g1p, be1p, w2d, b2p)


def _edge_pass_c(pre2, mom2a, mom2b, g2p, be2p, n):
    L, W = pre2.shape
    R2 = _EDGE_CHUNK2
    G = L // R2

    def body(p_ref, ma_ref, mb_ref, g_ref, be_ref, o_ref):
        s, t = _paired_scale(ma_ref[...] + mb_ref[...], g_ref[...],
                             be_ref[...], n)
        o_ref[...] = jnp.maximum(p_ref[...] * s + t, 0.0)

    small = pl.BlockSpec((1, W), lambda i: (0, 0))
    return pl.pallas_call(
        body,
        grid=(G,),
        in_specs=[
            pl.BlockSpec((R2, W), lambda i: (i, 0)),
            pl.BlockSpec((2, W), lambda i: (0, 0)),
            pl.BlockSpec((2, W), lambda i: (0, 0)),
            small, small,
        ],
        out_specs=pl.BlockSpec((R2, W), lambda i: (i, 0)),
        out_shape=jax.ShapeDtypeStruct((L, W), jnp.float32),
    )(pre2, mom2a, mom2b, g2p, be2p)


def _node_init(x, w_in, b_in, w1a, w1b):
    N = x.shape[0]
    H = w_in.shape[1]

    def body(x_ref, w_ref, b_ref, wa_ref, wb_ref, h_ref, p_ref, q_ref):
        h = jnp.dot(x_ref[...], w_ref[...],
                    preferred_element_type=jnp.float32) + b_ref[...]
        h_ref[...] = h
        p_ref[...] = jnp.dot(h, wa_ref[...], preferred_element_type=jnp.float32)
        q_ref[...] = jnp.dot(h, wb_ref[...], preferred_element_type=jnp.float32)

    return pl.pallas_call(
        body,
        out_shape=[jax.ShapeDtypeStruct((N, H), jnp.float32)] * 3,
    )(x, w_in, b_in.reshape(1, H), w1a, w1b)


def _node_update(h, parts, uw1a, uw1b, ub1, ug1, ube1, uw2, ub2,
                 ug2, ube2, w1a_next, w1b_next):
    N, H = h.shape

    def body(h_ref, p_ref, w1a_ref, w1b_ref, b1_ref, g1_ref, be1_ref,
             w2_ref, b2_ref, g2_ref, be2_ref, wan_ref, wbn_ref,
             o_ref, pn_ref, qn_ref):
        h_ = h_ref[...]
        aggr = p_ref[:N, :] + p_ref[N:, :]
        pre1 = (jnp.dot(h_, w1a_ref[...], preferred_element_type=jnp.float32)
                + jnp.dot(aggr, w1b_ref[...],
                          preferred_element_type=jnp.float32) + b1_ref[...])
        m1 = jnp.mean(pre1, axis=0, keepdims=True)
        v1 = jnp.mean((pre1 - m1) ** 2, axis=0, keepdims=True)
        h1 = jnp.maximum((pre1 - m1) * lax.rsqrt(v1 + _EPS) * g1_ref[...]
                         + be1_ref[...], 0.0)
        pre2 = jnp.dot(h1, w2_ref[...],
                       preferred_element_type=jnp.float32) + b2_ref[...]
        m2 = jnp.mean(pre2, axis=0, keepdims=True)
        v2 = jnp.mean((pre2 - m2) ** 2, axis=0, keepdims=True)
        upd = jnp.maximum((pre2 - m2) * lax.rsqrt(v2 + _EPS) * g2_ref[...]
                          + be2_ref[...], 0.0)
        h_new = h_ + upd
        o_ref[...] = h_new
        pn_ref[...] = jnp.dot(h_new, wan_ref[...],
                              preferred_element_type=jnp.float32)
        qn_ref[...] = jnp.dot(h_new, wbn_ref[...],
                              preferred_element_type=jnp.float32)

    return pl.pallas_call(
        body,
        out_shape=[jax.ShapeDtypeStruct((N, H), jnp.float32)] * 3,
    )(h, parts, uw1a, uw1b, ub1.reshape(1, H), ug1.reshape(1, H),
      ube1.reshape(1, H), uw2, ub2.reshape(1, H), ug2.reshape(1, H),
      ube2.reshape(1, H), w1a_next, w1b_next)


def _pool_project(h, batch, w_mu, b_mu, ng):
    N, H = h.shape
    LAT = w_mu.shape[1]

    def body(h_ref, b_ref, w_ref, bm_ref, o_ref):
        seg = b_ref[...]
        onehot = (seg == lax.broadcasted_iota(jnp.int32, (N, ng), 1)
                  ).astype(jnp.float32)
        sums = lax.dot_general(onehot, h_ref[...],
                               (((0,), (0,)), ((), ())),
                               preferred_element_type=jnp.float32)
        counts = jnp.sum(onehot, axis=0)[:, None]
        pooled = sums / jnp.maximum(counts, 1.0)
        o_ref[...] = jnp.dot(pooled, w_ref[...],
                             preferred_element_type=jnp.float32) + bm_ref[...]

    return pl.pallas_call(
        body,
        out_shape=jax.ShapeDtypeStruct((ng, LAT), jnp.float32),
    )(h, batch.reshape(N, 1), w_mu, b_mu.reshape(1, LAT))


def _pair(v):
    return jnp.concatenate([v, v]).reshape(1, -1)


@jax.jit
def kernel(x, edge_index, edge_attr, batch, params):
    N = x.shape[0]
    H = params['W_in'].shape[1]
    E = edge_index.shape[1]
    EDIM = edge_attr.shape[1]
    NG = 64
    src = edge_index[0]
    dst = edge_index[1]
    zeros_nh = jnp.zeros((N, H), jnp.float32)

    E2 = E // 2
    eap = jnp.pad(jnp.concatenate([edge_attr[:E2], edge_attr[E2:]], axis=1),
                  ((0, 0), (0, 2 * H - 2 * EDIM))).astype(jnp.bfloat16)

    layers = params['layers']
    l0 = layers[0]['msg']
    h, P, Q = _node_init(x, params['W_in'], params['b_in'],
                         l0['W1'][:H], l0['W1'][H:2 * H])

    for li, lay in enumerate(layers):
        mp = lay['msg']
        up = lay['upd']
        w1c = mp['W1'][2 * H:]
        w1cp = jnp.zeros((2 * H, 2 * H), jnp.float32)
        w1cp = w1cp.at[:EDIM, :H].set(w1c).at[EDIM:2 * EDIM, H:].set(w1c)
        w2d = jnp.zeros((2 * H, 2 * H), jnp.float32)
        w2d = w2d.at[:H, :H].set(mp['W2']).at[H:, H:].set(mp['W2'])

        mzero = jnp.zeros((2, 2 * H), jnp.float32)
        g2 = _sc_gather_sum(P, Q, dst, src)
        pre1, mom1 = _edge_pass_a(g2, eap,
                                  w1cp.astype(jnp.bfloat16),
                                  _pair(mp['b1']), 0)
        pre2, mom2 = _edge_pass_b(pre1, mom1, mzero, _pair(mp['g1']),
                                  _pair(mp['be1']),
                                  w2d.astype(jnp.bfloat16),
                                  _pair(mp['b2']), E)
        msg2 = _edge_pass_c(pre2, mom2, mzero, _pair(mp['g2']),
                            _pair(mp['be2']), E)
        parts = _sc_scatter_add(msg2, dst, zeros_nh, 0)
        if li + 1 < len(layers):
            nmp = layers[li + 1]['msg']
            wan, wbn = nmp['W1'][:H], nmp['W1'][H:2 * H]
        else:
            wan, wbn = up['W2'], up['W2']
        h, P, Q = _node_update(h, parts, up['W1'][:H],
                               up['W1'][H:2 * H], up['b1'], up['g1'],
                               up['be1'], up['W2'], up['b2'], up['g2'],
                               up['be2'], wan, wbn)

    return _pool_project(h, batch, params['W_mu'], params['b_mu'], NG)


# speedup vs baseline: 1.9191x; 1.0935x over previous
import functools

import jax
import jax.numpy as jnp
from jax import lax
from jax.experimental import pallas as pl
from jax.experimental.pallas import tpu as pltpu
from jax.experimental.pallas import tpu_sc as plsc

_EPS = 1e-5
_NC = 2
_NS = 16
_NW = _NC * _NS


def _sc_gather(P, Q, dst, src, row0, nrows):
    E = dst.shape[0]
    E2 = E // 2
    H = P.shape[1]
    ept = nrows // _NS
    C = 400 if ept % 400 == 0 else 200
    nch = ept // C
    mesh = plsc.VectorSubcoreMesh(core_axis_name="c", subcore_axis_name="s")

    @functools.partial(
        pl.kernel,
        out_type=(jax.ShapeDtypeStruct((nrows, 2 * H), jnp.float32),
                  jax.ShapeDtypeStruct((nrows, 2 * H), jnp.float32)),
        mesh=mesh,
        scratch_types=[
            pltpu.VMEM((C,), jnp.int32),
            pltpu.VMEM((C,), jnp.int32),
            pltpu.VMEM((C, H), jnp.float32),
            pltpu.VMEM((C, H), jnp.float32),
            pltpu.SemaphoreType.DMA,
            pltpu.SemaphoreType.DMA,
        ],
        compiler_params=pltpu.CompilerParams(use_tc_tiling_on_sc=False),
    )
    def k(p_hbm, q_hbm, dst_hbm, src_hbm, gp_hbm, gq_hbm,
          di, si, bp, bq, sem1, sem2):
        wid = lax.axis_index("s") * _NC + lax.axis_index("c")
        half = wid // _NS
        tw = wid % _NS
        col = half * H
        ebase = half * E2 + row0 + tw * ept
        rbase = tw * ept

        @pl.loop(0, nch)
        def _(c):
            off = ebase + c * C
            pltpu.sync_copy(dst_hbm.at[pl.ds(off, C)], di)
            pltpu.sync_copy(src_hbm.at[pl.ds(off, C)], si)
            cp1 = pltpu.async_copy(p_hbm.at[di], bp, sem1)
            cp2 = pltpu.async_copy(q_hbm.at[si], bq, sem2)
            cp1.wait()
            cp2.wait()
            row = rbase + c * C
            pltpu.sync_copy(bp, gp_hbm.at[pl.ds(row, C), pl.ds(col, H)])
            pltpu.sync_copy(bq, gq_hbm.at[pl.ds(row, C), pl.ds(col, H)])

    return k(P, Q, dst, src)


def _sc_gather_sum(P, Q, dst, src):
    E = dst.shape[0]
    E2 = E // 2
    H = P.shape[1]
    ept = E // _NW
    C = 400
    nch = ept // C
    mesh = plsc.VectorSubcoreMesh(core_axis_name="c", subcore_axis_name="s")

    @functools.partial(
        pl.kernel,
        out_type=jax.ShapeDtypeStruct((E2, 2 * H), jnp.float32),
        mesh=mesh,
        scratch_types=[
            pltpu.VMEM((C,), jnp.int32),
            pltpu.VMEM((C,), jnp.int32),
            pltpu.VMEM((C,), jnp.int32),
            pltpu.VMEM((C,), jnp.int32),
            pltpu.VMEM((C, H), jnp.float32),
            pltpu.VMEM((C, H), jnp.float32),
            pltpu.VMEM((C, H), jnp.float32),
            pltpu.VMEM((C, H), jnp.float32),
            pltpu.SemaphoreType.DMA,
            pltpu.SemaphoreType.DMA,
            pltpu.SemaphoreType.DMA,
        ],
        compiler_params=pltpu.CompilerParams(use_tc_tiling_on_sc=False),
    )
    def k(p_hbm, q_hbm, dst_hbm, src_hbm, g_hbm,
          di0, si0, di1, si1, a0, b0, a1, b1, sg0, sg1, sw):
        wid = lax.axis_index("s") * _NC + lax.axis_index("c")
        half = wid // _NS
        tw = wid % _NS
        col = half * H
        ebase = half * E2 + tw * ept
        rbase = tw * ept
        bufs = [(di0, si0, a0, b0, sg0), (di1, si1, a1, b1, sg1)]

        def issue(c):
            di, si, a, b, sg = bufs[c % 2]
            off = ebase + c * C
            pltpu.sync_copy(dst_hbm.at[pl.ds(off, C)], di)
            pltpu.sync_copy(src_hbm.at[pl.ds(off, C)], si)
            return (pltpu.async_copy(p_hbm.at[di], a, sg),
                    pltpu.async_copy(q_hbm.at[si], b, sg))

        pend = {0: issue(0), 1: issue(1)}
        wr = {}
        for c in range(nch):
            g1, g2 = pend.pop(c)
            g1.wait()
            g2.wait()
            _, _, a, b, _ = bufs[c % 2]

            @pl.loop(0, C)
            def _(r):
                for kk in range(H // 16):
                    sl = pl.ds(kk * 16, 16)
                    a[r, sl] = a[r, sl] + b[r, sl]

            wr[c] = pltpu.async_copy(
                a, g_hbm.at[pl.ds(rbase + c * C, C), pl.ds(col, H)], sw)
            nxt = c + 2
            if nxt < nch:
                wr.pop(c).wait()
                pend[nxt] = issue(nxt)
        for c in sorted(wr):
            wr.pop(c).wait()

    return k(P, Q, dst, src)


def _sc_scatter_add(msg2, dst, zeros_nh, row0):
    E = dst.shape[0]
    E2 = E // 2
    nrows = msg2.shape[0]
    N, H = zeros_nh.shape
    ept = nrows // _NS
    C = 400 if ept % 400 == 0 else 200
    nch = ept // C
    rpt = N // _NS
    mesh = plsc.VectorSubcoreMesh(core_axis_name="c", subcore_axis_name="s")

    @functools.partial(
        pl.kernel,
        out_type=jax.ShapeDtypeStruct((N, 2 * H), jnp.float32),
        mesh=mesh,
        scratch_types=[
            pltpu.VMEM((C,), jnp.int32),
            pltpu.VMEM((C,), jnp.int32),
            pltpu.VMEM((C, H), jnp.float32),
            pltpu.VMEM((C, H), jnp.float32),
            pltpu.VMEM((rpt // 5, H), jnp.float32),
            pltpu.VMEM_SHARED((N, H), jnp.float32),
            pltpu.SemaphoreType.DMA,
            pltpu.SemaphoreType.DMA,
            pltpu.SemaphoreType.DMA,
        ],
        compiler_params=pltpu.CompilerParams(use_tc_tiling_on_sc=False),
    )
    def k(msg_hbm, dst_hbm, z_hbm, out_hbm, di0, di1, m0, m1, stage,
          acc_sh, sm0, sm1, ss):
        cid = lax.axis_index("c")
        sid = lax.axis_index("s")
        r0 = sid * rpt
        rs = rpt // 5
        for j in range(5):
            pltpu.sync_copy(z_hbm.at[pl.ds(r0 + j * rs, rs)], stage)
            pltpu.sync_copy(stage, acc_sh.at[pl.ds(r0 + j * rs, rs)])
        plsc.subcore_barrier()

        col = cid * H
        ebase = cid * E2 + row0 + sid * ept
        rbase = sid * ept
        bufs = [(di0, m0, sm0), (di1, m1, sm1)]

        def issue(c):
            di, m, sm = bufs[c % 2]
            pltpu.sync_copy(dst_hbm.at[pl.ds(ebase + c * C, C)], di)
            return pltpu.async_copy(
                msg_hbm.at[pl.ds(rbase + c * C, C), pl.ds(col, H)], m, sm)

        pend = {0: issue(0), 1: issue(1)}
        for c in range(nch):
            pend.pop(c).wait()
            di, m, _ = bufs[c % 2]
            pltpu.sync_copy(m, acc_sh.at[di], add=True)
            nxt = c + 2
            if nxt < nch:
                pend[nxt] = issue(nxt)

        plsc.subcore_barrier()
        for j in range(5):
            pltpu.sync_copy(acc_sh.at[pl.ds(r0 + j * rs, rs)], stage)
            pltpu.sync_copy(stage,
                            out_hbm.at[pl.ds(r0 + j * rs, rs), pl.ds(col, H)])

    return k(msg2, dst, zeros_nh)


_EDGE_CHUNK2 = 4000


def _edge_pass_a(g2, eap, w1cp, b1p, row0):
    L, W = g2.shape
    R2 = _EDGE_CHUNK2
    G = L // R2
    blk0 = row0 // R2

    def body(g_ref, ea_ref, w_ref, b_ref, pre_ref, mom_ref):
        i = pl.program_id(0)
        pre = g_ref[...] + b_ref[...] + jnp.dot(
            ea_ref[...], w_ref[...], preferred_element_type=jnp.float32)
        pre_ref[...] = pre
        mom = jnp.stack([jnp.sum(pre, axis=0), jnp.sum(pre * pre, axis=0)])
        mom_ref[...] = jnp.where(i == 0, 0.0, mom_ref[...]) + mom

    return pl.pallas_call(
        body,
        grid=(G,),
        in_specs=[
            pl.BlockSpec((R2, W), lambda i: (i, 0)),
            pl.BlockSpec((R2, W), lambda i: (i + blk0, 0)),
            pl.BlockSpec((W, W), lambda i: (0, 0)),
            pl.BlockSpec((1, W), lambda i: (0, 0)),
        ],
        out_specs=[
            pl.BlockSpec((R2, W), lambda i: (i, 0)),
            pl.BlockSpec((2, W), lambda i: (0, 0)),
        ],
        out_shape=[
            jax.ShapeDtypeStruct((L, W), jnp.float32),
            jax.ShapeDtypeStruct((2, W), jnp.float32),
        ],
    )(g2, eap, w1cp, b1p)


def _paired_scale(mom, gp, bep, n):
    H = mom.shape[1] // 2
    tot = mom[:, :H] + mom[:, H:]
    m = tot[0:1, :] * (1.0 / n)
    v = tot[1:2, :] * (1.0 / n) - m * m
    m2 = jnp.concatenate([m, m], axis=1)
    v2 = jnp.concatenate([v, v], axis=1)
    s = gp * lax.rsqrt(v2 + _EPS)
    t = bep - m2 * s
    return s, t


def _edge_pass_b(pre1, mom1a, mom1b, g1p, be1p, w2d, b2p, n):
    L, W = pre1.shape
    R2 = _EDGE_CHUNK2
    G = L // R2

    def body(p_ref, ma_ref, mb_ref, g_ref, be_ref, w_ref, b_ref,
             o_ref, mom_ref):
        i = pl.program_id(0)
        s, t = _paired_scale(ma_ref[...] + mb_ref[...], g_ref[...],
                             be_ref[...], n)
        h1 = jnp.maximum(p_ref[...] * s + t, 0.0)
        pre2 = jnp.dot(h1.astype(jnp.bfloat16), w_ref[...],
                       preferred_element_type=jnp.float32) + b_ref[...]
        o_ref[...] = pre2
        mom = jnp.stack([jnp.sum(pre2, axis=0), jnp.sum(pre2 * pre2, axis=0)])
        mom_ref[...] = jnp.where(i == 0, 0.0, mom_ref[...]) + mom

    small = pl.BlockSpec((1, W), lambda i: (0, 0))
    return pl.pallas_call(
        body,
        grid=(G,),
        in_specs=[
            pl.BlockSpec((R2, W), lambda i: (i, 0)),
            pl.BlockSpec((2, W), lambda i: (0, 0)),
            pl.BlockSpec((2, W), lambda i: (0, 0)),
            small, small,
            pl.BlockSpec((W, W), lambda i: (0, 0)),
            small,
        ],
        out_specs=[
            pl.BlockSpec((R2, W), lambda i: (i, 0)),
            pl.BlockSpec((2, W), lambda i: (0, 0)),
        ],
        out_shape=[
            jax.ShapeDtypeStruct((L, W), jnp.float32),
            jax.ShapeDtypeStruct((2, W), jnp.float32),
        ],
    )(pre1, mom1a, mom1b, g1p, be1p, w2d, b2p)


def _edge_pass_c(pre2, mom2a, mom2b, g2p, be2p, n):
    L, W = pre2.shape
    R2 = _EDGE_CHUNK2
    G = L // R2

    def body(p_ref, ma_ref, mb_ref, g_ref, be_ref, o_ref):
        s, t = _paired_scale(ma_ref[...] + mb_ref[...], g_ref[...],
                             be_ref[...], n)
        o_ref[...] = jnp.maximum(p_ref[...] * s + t, 0.0)

    small = pl.BlockSpec((1, W), lambda i: (0, 0))
    return pl.pallas_call(
        body,
        grid=(G,),
        in_specs=[
            pl.BlockSpec((R2, W), lambda i: (i, 0)),
            pl.BlockSpec((2, W), lambda i: (0, 0)),
            pl.BlockSpec((2, W), lambda i: (0, 0)),
            small, small,
        ],
        out_specs=pl.BlockSpec((R2, W), lambda i: (i, 0)),
        out_shape=jax.ShapeDtypeStruct((L, W), jnp.float32),
    )(pre2, mom2a, mom2b, g2p, be2p)


def _node_init(x, w_in, b_in, w1a, w1b):
    N = x.shape[0]
    H = w_in.shape[1]

    def body(x_ref, w_ref, b_ref, wa_ref, wb_ref, h_ref, p_ref, q_ref):
        h = jnp.dot(x_ref[...], w_ref[...],
                    preferred_element_type=jnp.float32) + b_ref[...]
        h_ref[...] = h
        p_ref[...] = jnp.dot(h, wa_ref[...], preferred_element_type=jnp.float32)
        q_ref[...] = jnp.dot(h, wb_ref[...], preferred_element_type=jnp.float32)

    return pl.pallas_call(
        body,
        out_shape=[jax.ShapeDtypeStruct((N, H), jnp.float32)] * 3,
    )(x, w_in, b_in.reshape(1, H), w1a, w1b)


def _node_update(h, parts, uw1a, uw1b, ub1, ug1, ube1, uw2, ub2,
                 ug2, ube2, w1a_next, w1b_next):
    N, H = h.shape

    def body(h_ref, p_ref, w1a_ref, w1b_ref, b1_ref, g1_ref, be1_ref,
             w2_ref, b2_ref, g2_ref, be2_ref, wan_ref, wbn_ref,
             o_ref, pn_ref, qn_ref):
        h_ = h_ref[...]
        aggr = p_ref[:, :H] + p_ref[:, H:]
        pre1 = (jnp.dot(h_, w1a_ref[...], preferred_element_type=jnp.float32)
                + jnp.dot(aggr, w1b_ref[...],
                          preferred_element_type=jnp.float32) + b1_ref[...])
        m1 = jnp.mean(pre1, axis=0, keepdims=True)
        v1 = jnp.mean((pre1 - m1) ** 2, axis=0, keepdims=True)
        h1 = jnp.maximum((pre1 - m1) * lax.rsqrt(v1 + _EPS) * g1_ref[...]
                         + be1_ref[...], 0.0)
        pre2 = jnp.dot(h1, w2_ref[...],
                       preferred_element_type=jnp.float32) + b2_ref[...]
        m2 = jnp.mean(pre2, axis=0, keepdims=True)
        v2 = jnp.mean((pre2 - m2) ** 2, axis=0, keepdims=True)
        upd = jnp.maximum((pre2 - m2) * lax.rsqrt(v2 + _EPS) * g2_ref[...]
                          + be2_ref[...], 0.0)
        h_new = h_ + upd
        o_ref[...] = h_new
        pn_ref[...] = jnp.dot(h_new, wan_ref[...],
                              preferred_element_type=jnp.float32)
        qn_ref[...] = jnp.dot(h_new, wbn_ref[...],
                              preferred_element_type=jnp.float32)

    return pl.pallas_call(
        body,
        out_shape=[jax.ShapeDtypeStruct((N, H), jnp.float32)] * 3,
    )(h, parts, uw1a, uw1b, ub1.reshape(1, H), ug1.reshape(1, H),
      ube1.reshape(1, H), uw2, ub2.reshape(1, H), ug2.reshape(1, H),
      ube2.reshape(1, H), w1a_next, w1b_next)


def _pool_project(h, batch, w_mu, b_mu, ng):
    N, H = h.shape
    LAT = w_mu.shape[1]

    def body(h_ref, b_ref, w_ref, bm_ref, o_ref):
        seg = b_ref[...]
        onehot = (seg == lax.broadcasted_iota(jnp.int32, (N, ng), 1)
                  ).astype(jnp.float32)
        sums = lax.dot_general(onehot, h_ref[...],
                               (((0,), (0,)), ((), ())),
                               preferred_element_type=jnp.float32)
        counts = jnp.sum(onehot, axis=0)[:, None]
        pooled = sums / jnp.maximum(counts, 1.0)
        o_ref[...] = jnp.dot(pooled, w_ref[...],
                             preferred_element_type=jnp.float32) + bm_ref[...]

    return pl.pallas_call(
        body,
        out_shape=jax.ShapeDtypeStruct((ng, LAT), jnp.float32),
    )(h, batch.reshape(N, 1), w_mu, b_mu.reshape(1, LAT))


def _pair(v):
    return jnp.concatenate([v, v]).reshape(1, -1)


@jax.jit
def kernel(x, edge_index, edge_attr, batch, params):
    N = x.shape[0]
    H = params['W_in'].shape[1]
    E = edge_index.shape[1]
    EDIM = edge_attr.shape[1]
    NG = 64
    src = edge_index[0]
    dst = edge_index[1]
    zeros_nh = jnp.zeros((N, H), jnp.float32)

    E2 = E // 2
    eap = jnp.pad(jnp.concatenate([edge_attr[:E2], edge_attr[E2:]], axis=1),
                  ((0, 0), (0, 2 * H - 2 * EDIM))).astype(jnp.bfloat16)

    layers = params['layers']
    l0 = layers[0]['msg']
    h, P, Q = _node_init(x, params['W_in'], params['b_in'],
                         l0['W1'][:H], l0['W1'][H:2 * H])

    for li, lay in enumerate(layers):
        mp = lay['msg']
        up = lay['upd']
        w1c = mp['W1'][2 * H:]
        w1cp = jnp.zeros((2 * H, 2 * H), jnp.float32)
        w1cp = w1cp.at[:EDIM, :H].set(w1c).at[EDIM:2 * EDIM, H:].set(w1c)
        w2d = jnp.zeros((2 * H, 2 * H), jnp.float32)
        w2d = w2d.at[:H, :H].set(mp['W2']).at[H:, H:].set(mp['W2'])

        mzero = jnp.zeros((2, 2 * H), jnp.float32)
        g2 = _sc_gather_sum(P, Q, dst, src)
        pre1, mom1 = _edge_pass_a(g2, eap,
                                  w1cp.astype(jnp.bfloat16),
                                  _pair(mp['b1']), 0)
        pre2, mom2 = _edge_pass_b(pre1, mom1, mzero, _pair(mp['g1']),
                                  _pair(mp['be1']),
                                  w2d.astype(jnp.bfloat16),
                                  _pair(mp['b2']), E)
        msg2 = _edge_pass_c(pre2, mom2, mzero, _pair(mp['g2']),
                            _pair(mp['be2']), E)
        parts = _sc_scatter_add(msg2, dst, zeros_nh, 0)
        if li + 1 < len(layers):
            nmp = layers[li + 1]['msg']
            wan, wbn = nmp['W1'][:H], nmp['W1'][H:2 * H]
        else:
            wan, wbn = up['W2'], up['W2']
        h, P, Q = _node_update(h, parts, up['W1'][:H],
                               up['W1'][H:2 * H], up['b1'], up['g1'],
                               up['be1'], up['W2'], up['b2'], up['g2'],
                               up['be2'], wan, wbn)

    return _pool_project(h, batch, params['W_mu'], params['b_mu'], NG)


# speedup vs baseline: 1.9334x; 1.0074x over previous
import functools

import jax
import jax.numpy as jnp
from jax import lax
from jax.experimental import pallas as pl
from jax.experimental.pallas import tpu as pltpu
from jax.experimental.pallas import tpu_sc as plsc

_EPS = 1e-5
_NC = 2
_NS = 16
_NW = _NC * _NS


def _sc_gather_sum(P, Q, dst, src):
    E = dst.shape[0]
    E2 = E // 2
    H = P.shape[1]
    ept = E // _NW
    C = 400
    nch = ept // C
    mesh = plsc.VectorSubcoreMesh(core_axis_name="c", subcore_axis_name="s")

    @functools.partial(
        pl.kernel,
        out_type=jax.ShapeDtypeStruct((E2, 2 * H), jnp.float32),
        mesh=mesh,
        scratch_types=[
            pltpu.VMEM((C,), jnp.int32),
            pltpu.VMEM((C,), jnp.int32),
            pltpu.VMEM((C,), jnp.int32),
            pltpu.VMEM((C,), jnp.int32),
            pltpu.VMEM((C, H), jnp.float32),
            pltpu.VMEM((C, H), jnp.float32),
            pltpu.VMEM((C, H), jnp.float32),
            pltpu.VMEM((C, H), jnp.float32),
            pltpu.SemaphoreType.DMA,
            pltpu.SemaphoreType.DMA,
            pltpu.SemaphoreType.DMA,
        ],
        compiler_params=pltpu.CompilerParams(use_tc_tiling_on_sc=False),
    )
    def k(p_hbm, q_hbm, dst_hbm, src_hbm, g_hbm,
          di0, si0, di1, si1, a0, b0, a1, b1, sg0, sg1, sw):
        wid = lax.axis_index("s") * _NC + lax.axis_index("c")
        half = wid // _NS
        tw = wid % _NS
        col = half * H
        ebase = half * E2 + tw * ept
        rbase = tw * ept
        bufs = [(di0, si0, a0, b0, sg0), (di1, si1, a1, b1, sg1)]

        def issue(c):
            di, si, a, b, sg = bufs[c % 2]
            off = ebase + c * C
            pltpu.sync_copy(dst_hbm.at[pl.ds(off, C)], di)
            pltpu.sync_copy(src_hbm.at[pl.ds(off, C)], si)
            return (pltpu.async_copy(p_hbm.at[di], a, sg),
                    pltpu.async_copy(q_hbm.at[si], b, sg))

        pend = {0: issue(0), 1: issue(1)}
        wr = {}
        for c in range(nch):
            g1, g2 = pend.pop(c)
            g1.wait()
            g2.wait()
            _, _, a, b, _ = bufs[c % 2]

            @pl.loop(0, C)
            def _(r):
                for kk in range(H // 16):
                    sl = pl.ds(kk * 16, 16)
                    a[r, sl] = a[r, sl] + b[r, sl]

            wr[c] = pltpu.async_copy(
                a, g_hbm.at[pl.ds(rbase + c * C, C), pl.ds(col, H)], sw)
            nxt = c + 2
            if nxt < nch:
                wr.pop(c).wait()
                pend[nxt] = issue(nxt)
        for c in sorted(wr):
            wr.pop(c).wait()

    return k(P, Q, dst, src)


def _sc_scatter_add(msg2, st2, dst, zeros_nh, row0):
    E = dst.shape[0]
    E2 = E // 2
    nrows = msg2.shape[0]
    N, H = zeros_nh.shape
    ept = nrows // _NS
    C = 400 if ept % 400 == 0 else 200
    nch = ept // C
    rpt = N // _NS
    mesh = plsc.VectorSubcoreMesh(core_axis_name="c", subcore_axis_name="s")

    @functools.partial(
        pl.kernel,
        out_type=jax.ShapeDtypeStruct((N, 2 * H), jnp.float32),
        mesh=mesh,
        scratch_types=[
            pltpu.VMEM((C,), jnp.int32),
            pltpu.VMEM((C,), jnp.int32),
            pltpu.VMEM((C, H), jnp.float32),
            pltpu.VMEM((C, H), jnp.float32),
            pltpu.VMEM((2, 2 * H), jnp.float32),
            pltpu.VMEM((rpt // 5, H), jnp.float32),
            pltpu.VMEM_SHARED((N, H), jnp.float32),
            pltpu.SemaphoreType.DMA,
            pltpu.SemaphoreType.DMA,
            pltpu.SemaphoreType.DMA,
        ],
        compiler_params=pltpu.CompilerParams(use_tc_tiling_on_sc=False),
    )
    def k(msg_hbm, st_hbm, dst_hbm, z_hbm, out_hbm, di0, di1, m0, m1, stb,
          stage, acc_sh, sm0, sm1, ss):
        cid = lax.axis_index("c")
        sid = lax.axis_index("s")
        pltpu.sync_copy(st_hbm, stb)
        r0 = sid * rpt
        rs = rpt // 5
        for j in range(5):
            pltpu.sync_copy(z_hbm.at[pl.ds(r0 + j * rs, rs)], stage)
            pltpu.sync_copy(stage, acc_sh.at[pl.ds(r0 + j * rs, rs)])
        plsc.subcore_barrier()

        col = cid * H
        ebase = cid * E2 + row0 + sid * ept
        rbase = sid * ept
        bufs = [(di0, m0, sm0), (di1, m1, sm1)]

        def issue(c):
            di, m, sm = bufs[c % 2]
            pltpu.sync_copy(dst_hbm.at[pl.ds(ebase + c * C, C)], di)
            return pltpu.async_copy(
                msg_hbm.at[pl.ds(rbase + c * C, C), pl.ds(col, H)], m, sm)

        sk = [stb[0, pl.ds(col + 16 * k, 16)] for k in range(H // 16)]
        tk = [stb[1, pl.ds(col + 16 * k, 16)] for k in range(H // 16)]

        pend = {0: issue(0), 1: issue(1)}
        for c in range(nch):
            pend.pop(c).wait()
            di, m, _ = bufs[c % 2]

            @pl.loop(0, C)
            def _(r):
                for k in range(H // 16):
                    sl = pl.ds(16 * k, 16)
                    m[r, sl] = jnp.maximum(m[r, sl] * sk[k] + tk[k], 0.0)

            pltpu.sync_copy(m, acc_sh.at[di], add=True)
            nxt = c + 2
            if nxt < nch:
                pend[nxt] = issue(nxt)

        plsc.subcore_barrier()
        for j in range(5):
            pltpu.sync_copy(acc_sh.at[pl.ds(r0 + j * rs, rs)], stage)
            pltpu.sync_copy(stage,
                            out_hbm.at[pl.ds(r0 + j * rs, rs), pl.ds(col, H)])

    return k(msg2, st2, dst, zeros_nh)


_EDGE_CHUNK2 = 4000


def _edge_pass_a(g2, eap, w1cp, b1p, row0):
    L, W = g2.shape
    R2 = _EDGE_CHUNK2
    G = L // R2
    blk0 = row0 // R2

    def body(g_ref, ea_ref, w_ref, b_ref, pre_ref, mom_ref):
        i = pl.program_id(0)
        pre = g_ref[...] + b_ref[...] + jnp.dot(
            ea_ref[...], w_ref[...], preferred_element_type=jnp.float32)
        pre_ref[...] = pre
        mom = jnp.stack([jnp.sum(pre, axis=0), jnp.sum(pre * pre, axis=0)])
        mom_ref[...] = jnp.where(i == 0, 0.0, mom_ref[...]) + mom

    return pl.pallas_call(
        body,
        grid=(G,),
        in_specs=[
            pl.BlockSpec((R2, W), lambda i: (i, 0)),
            pl.BlockSpec((R2, W), lambda i: (i + blk0, 0)),
            pl.BlockSpec((W, W), lambda i: (0, 0)),
            pl.BlockSpec((1, W), lambda i: (0, 0)),
        ],
        out_specs=[
            pl.BlockSpec((R2, W), lambda i: (i, 0)),
            pl.BlockSpec((2, W), lambda i: (0, 0)),
        ],
        out_shape=[
            jax.ShapeDtypeStruct((L, W), jnp.float32),
            jax.ShapeDtypeStruct((2, W), jnp.float32),
        ],
    )(g2, eap, w1cp, b1p)


def _paired_scale(mom, gp, bep, n):
    H = mom.shape[1] // 2
    tot = mom[:, :H] + mom[:, H:]
    m = tot[0:1, :] * (1.0 / n)
    v = tot[1:2, :] * (1.0 / n) - m * m
    m2 = jnp.concatenate([m, m], axis=1)
    v2 = jnp.concatenate([v, v], axis=1)
    s = gp * lax.rsqrt(v2 + _EPS)
    t = bep - m2 * s
    return s, t


def _edge_pass_b(pre1, mom1a, mom1b, g1p, be1p, w2d, b2p, g2p, be2p, n):
    L, W = pre1.shape
    R2 = _EDGE_CHUNK2
    G = L // R2

    def body(p_ref, ma_ref, mb_ref, g_ref, be_ref, w_ref, b_ref,
             g2_ref, be2_ref, o_ref, st_ref, mom_ref):
        i = pl.program_id(0)
        s, t = _paired_scale(ma_ref[...] + mb_ref[...], g_ref[...],
                             be_ref[...], n)
        h1 = jnp.maximum(p_ref[...] * s + t, 0.0)
        pre2 = jnp.dot(h1.astype(jnp.bfloat16), w_ref[...],
                       preferred_element_type=jnp.float32) + b_ref[...]
        o_ref[...] = pre2
        mom = jnp.stack([jnp.sum(pre2, axis=0), jnp.sum(pre2 * pre2, axis=0)])
        newmom = jnp.where(i == 0, 0.0, mom_ref[...]) + mom
        mom_ref[...] = newmom

        @pl.when(i == G - 1)
        def _():
            s2, t2 = _paired_scale(newmom, g2_ref[...], be2_ref[...], n)
            st_ref[...] = jnp.concatenate([s2, t2], axis=0)

    small = pl.BlockSpec((1, W), lambda i: (0, 0))
    return pl.pallas_call(
        body,
        grid=(G,),
        in_specs=[
            pl.BlockSpec((R2, W), lambda i: (i, 0)),
            pl.BlockSpec((2, W), lambda i: (0, 0)),
            pl.BlockSpec((2, W), lambda i: (0, 0)),
            small, small,
            pl.BlockSpec((W, W), lambda i: (0, 0)),
            small, small, small,
        ],
        out_specs=[
            pl.BlockSpec((R2, W), lambda i: (i, 0)),
            pl.BlockSpec((2, W), lambda i: (0, 0)),
            pl.BlockSpec((2, W), lambda i: (0, 0)),
        ],
        out_shape=[
            jax.ShapeDtypeStruct((L, W), jnp.float32),
            jax.ShapeDtypeStruct((2, W), jnp.float32),
            jax.ShapeDtypeStruct((2, W), jnp.float32),
        ],
    )(pre1, mom1a, mom1b, g1p, be1p, w2d, b2p, g2p, be2p)


def _node_init(x, w_in, b_in, w1a, w1b):
    N = x.shape[0]
    H = w_in.shape[1]

    def body(x_ref, w_ref, b_ref, wa_ref, wb_ref, h_ref, p_ref, q_ref):
        h = jnp.dot(x_ref[...], w_ref[...],
                    preferred_element_type=jnp.float32) + b_ref[...]
        h_ref[...] = h
        p_ref[...] = jnp.dot(h, wa_ref[...], preferred_element_type=jnp.float32)
        q_ref[...] = jnp.dot(h, wb_ref[...], preferred_element_type=jnp.float32)

    return pl.pallas_call(
        body,
        out_shape=[jax.ShapeDtypeStruct((N, H), jnp.float32)] * 3,
    )(x, w_in, b_in.reshape(1, H), w1a, w1b)


def _node_update(h, parts, uw1a, uw1b, ub1, ug1, ube1, uw2, ub2,
                 ug2, ube2, w1a_next, w1b_next):
    N, H = h.shape

    def body(h_ref, p_ref, w1a_ref, w1b_ref, b1_ref, g1_ref, be1_ref,
             w2_ref, b2_ref, g2_ref, be2_ref, wan_ref, wbn_ref,
             o_ref, pn_ref, qn_ref):
        h_ = h_ref[...]
        aggr = p_ref[:, :H] + p_ref[:, H:]
        pre1 = (jnp.dot(h_, w1a_ref[...], preferred_element_type=jnp.float32)
                + jnp.dot(aggr, w1b_ref[...],
                          preferred_element_type=jnp.float32) + b1_ref[...])
        m1 = jnp.mean(pre1, axis=0, keepdims=True)
        v1 = jnp.mean((pre1 - m1) ** 2, axis=0, keepdims=True)
        h1 = jnp.maximum((pre1 - m1) * lax.rsqrt(v1 + _EPS) * g1_ref[...]
                         + be1_ref[...], 0.0)
        pre2 = jnp.dot(h1, w2_ref[...],
                       preferred_element_type=jnp.float32) + b2_ref[...]
        m2 = jnp.mean(pre2, axis=0, keepdims=True)
        v2 = jnp.mean((pre2 - m2) ** 2, axis=0, keepdims=True)
        upd = jnp.maximum((pre2 - m2) * lax.rsqrt(v2 + _EPS) * g2_ref[...]
                          + be2_ref[...], 0.0)
        h_new = h_ + upd
        o_ref[...] = h_new
        pn_ref[...] = jnp.dot(h_new, wan_ref[...],
                              preferred_element_type=jnp.float32)
        qn_ref[...] = jnp.dot(h_new, wbn_ref[...],
                              preferred_element_type=jnp.float32)

    return pl.pallas_call(
        body,
        out_shape=[jax.ShapeDtypeStruct((N, H), jnp.float32)] * 3,
    )(h, parts, uw1a, uw1b, ub1.reshape(1, H), ug1.reshape(1, H),
      ube1.reshape(1, H), uw2, ub2.reshape(1, H), ug2.reshape(1, H),
      ube2.reshape(1, H), w1a_next, w1b_next)


def _pool_project(h, batch, w_mu, b_mu, ng):
    N, H = h.shape
    LAT = w_mu.shape[1]

    def body(h_ref, b_ref, w_ref, bm_ref, o_ref):
        seg = b_ref[...]
        onehot = (seg == lax.broadcasted_iota(jnp.int32, (N, ng), 1)
                  ).astype(jnp.float32)
        sums = lax.dot_general(onehot, h_ref[...],
                               (((0,), (0,)), ((), ())),
                               preferred_element_type=jnp.float32)
        counts = jnp.sum(onehot, axis=0)[:, None]
        pooled = sums / jnp.maximum(counts, 1.0)
        o_ref[...] = jnp.dot(pooled, w_ref[...],
                             preferred_element_type=jnp.float32) + bm_ref[...]

    return pl.pallas_call(
        body,
        out_shape=jax.ShapeDtypeStruct((ng, LAT), jnp.float32),
    )(h, batch.reshape(N, 1), w_mu, b_mu.reshape(1, LAT))


def _pair(v):
    return jnp.concatenate([v, v]).reshape(1, -1)


@jax.jit
def kernel(x, edge_index, edge_attr, batch, params):
    N = x.shape[0]
    H = params['W_in'].shape[1]
    E = edge_index.shape[1]
    EDIM = edge_attr.shape[1]
    NG = 64
    src = edge_index[0]
    dst = edge_index[1]
    zeros_nh = jnp.zeros((N, H), jnp.float32)

    E2 = E // 2
    eap = jnp.pad(jnp.concatenate([edge_attr[:E2], edge_attr[E2:]], axis=1),
                  ((0, 0), (0, 2 * H - 2 * EDIM))).astype(jnp.bfloat16)

    layers = params['layers']
    l0 = layers[0]['msg']
    h, P, Q = _node_init(x, params['W_in'], params['b_in'],
                         l0['W1'][:H], l0['W1'][H:2 * H])

    for li, lay in enumerate(layers):
        mp = lay['msg']
        up = lay['upd']
        w1c = mp['W1'][2 * H:]
        w1cp = jnp.zeros((2 * H, 2 * H), jnp.float32)
        w1cp = w1cp.at[:EDIM, :H].set(w1c).at[EDIM:2 * EDIM, H:].set(w1c)
        w2d = jnp.zeros((2 * H, 2 * H), jnp.float32)
        w2d = w2d.at[:H, :H].set(mp['W2']).at[H:, H:].set(mp['W2'])

        mzero = jnp.zeros((2, 2 * H), jnp.float32)
        g2 = _sc_gather_sum(P, Q, dst, src)
        pre1, mom1 = _edge_pass_a(g2, eap,
                                  w1cp.astype(jnp.bfloat16),
                                  _pair(mp['b1']), 0)
        pre2, st2, _ = _edge_pass_b(pre1, mom1, mzero, _pair(mp['g1']),
                                    _pair(mp['be1']),
                                    w2d.astype(jnp.bfloat16),
                                    _pair(mp['b2']), _pair(mp['g2']),
                                    _pair(mp['be2']), E)
        parts = _sc_scatter_add(pre2, st2, dst, zeros_nh, 0)
        if li + 1 < len(layers):
            nmp = layers[li + 1]['msg']
            wan, wbn = nmp['W1'][:H], nmp['W1'][H:2 * H]
        else:
            wan, wbn = up['W2'], up['W2']
        h, P, Q = _node_update(h, parts, up['W1'][:H],
                               up['W1'][H:2 * H], up['b1'], up['g1'],
                               up['be1'], up['W2'], up['b2'], up['g2'],
                               up['be2'], wan, wbn)

    return _pool_project(h, batch, params['W_mu'], params['b_mu'], NG)
